# Initial kernel scaffold; baseline (speedup 1.0000x reference)
import jax
import jax.numpy as jnp
from jax.experimental import pallas as pl


def kernel(y_pred):
    raise NotImplementedError("write your pallas kernel here")


# speedup vs baseline: 13.0171x; 13.0171x over previous
import functools

import jax
import jax.numpy as jnp
from jax.experimental import pallas as pl
from jax.experimental.pallas import tpu as pltpu

ROWS = 128
COLS = 32768
K = 64
BLOCK_ROWS = 8
SIGN = -(2 ** 31)


def _loss_block_kernel(x_ref, out_ref):
    i = pl.program_id(0)
    x = x_ref[...]

    col = jax.lax.broadcasted_iota(jnp.int32, x.shape, 1)
    valid = col >= 1

    b = jax.lax.bitcast_convert_type(x, jnp.int32)
    key = jnp.where(b < 0, b ^ jnp.int32(0x7FFFFFFF), b)
    key = jnp.where(valid, key, jnp.int32(SIGN))

    def body(it, t):
        bit = jnp.left_shift(jnp.int32(1), 31 - it)
        cand = t | bit
        scand = cand ^ jnp.int32(SIGN)
        cnt = jnp.sum((key >= scand).astype(jnp.int32), axis=1, keepdims=True)
        return jnp.where(cnt >= K, cand, t)

    t0 = jnp.zeros((x.shape[0], 1), jnp.int32)
    t = jax.lax.fori_loop(0, 32, body, t0)
    t_signed = t ^ jnp.int32(SIGN)

    b_t = jnp.where(t_signed < 0, t_signed ^ jnp.int32(0x7FFFFFFF), t_signed)
    x_t = jax.lax.bitcast_convert_type(b_t, jnp.float32)
    exp_t = jnp.exp(x_t)

    e = jnp.where(valid, jnp.exp(x), 0.0)
    total = jnp.sum(e, axis=1, keepdims=True)
    gt = key > t_signed
    cnt_gt = jnp.sum(gt.astype(jnp.int32), axis=1, keepdims=True)
    sum_gt = jnp.sum(jnp.where(gt, e, 0.0), axis=1, keepdims=True)

    top_sum = sum_gt + (K - cnt_gt).astype(jnp.float32) * exp_t
    ng = total - top_sum
    pos = x[:, 0:1]
    block_loss = jnp.sum(jnp.log(ng) - pos) * (1.0 / ROWS)

    @pl.when(i == 0)
    def _init():
        out_ref[0, 0] = 0.0

    out_ref[0, 0] += block_loss


@jax.jit
def kernel(y_pred):
    grid = (ROWS // BLOCK_ROWS,)
    out = pl.pallas_call(
        _loss_block_kernel,
        grid=grid,
        in_specs=[pl.BlockSpec((BLOCK_ROWS, COLS), lambda i: (i, 0))],
        out_specs=pl.BlockSpec(
            (1, 1), lambda i: (0, 0), memory_space=pltpu.SMEM
        ),
        out_shape=jax.ShapeDtypeStruct((1, 1), jnp.float32),
    )(y_pred)
    return out[0, 0]
